# baseline (speedup 1.0000x reference)
import functools

import jax
import jax.numpy as jnp
from jax import lax
from jax.experimental import pallas as pl
from jax.experimental.pallas import tpu as pltpu
from jax.experimental.pallas import tpu_sc as plsc

N = 10000
E = 320000
D = 128

NC = 2
NS = 16
NW = NC * NS
E_PER_W = E // NW
CHUNK = 80
N_CHUNKS = E_PER_W // CHUNK
N_PAIRS = N_CHUNKS // 2
WB_TILES = 10
WB_ROWS = N // WB_TILES


def _sc_spmm(packed3, val3, embeds, zeros_blk):
    mesh = plsc.VectorSubcoreMesh(core_axis_name="c", subcore_axis_name="s")

    @functools.partial(
        pl.kernel,
        out_type=jax.ShapeDtypeStruct((NC, N, D), jnp.float32),
        mesh=mesh,
        scratch_types=[
            pltpu.VMEM_SHARED((N, D), jnp.float32),
            pltpu.VMEM((E_PER_W,), jnp.int32),
            pltpu.VMEM((E_PER_W,), jnp.float32),
            pltpu.VMEM((CHUNK,), jnp.int32),
            pltpu.VMEM((CHUNK,), jnp.int32),
            pltpu.VMEM((CHUNK,), jnp.int32),
            pltpu.VMEM((CHUNK,), jnp.int32),
            pltpu.VMEM((CHUNK, D), jnp.float32),
            pltpu.VMEM((CHUNK, D), jnp.float32),
            pltpu.SemaphoreType.DMA,
            pltpu.SemaphoreType.DMA,
            pltpu.SemaphoreType.DMA,
            pltpu.SemaphoreType.DMA,
            pltpu.SemaphoreType.DMA,
            pltpu.SemaphoreType.DMA,
        ],
    )
    def k(packed_h, val_h, emb_h, zero_h, out_h,
          acc, packed, vals, colb0, colb1, rowb0, rowb1, buf0, buf1,
          gs0a, gs0b, gs1a, gs1b, ss0, ss1):
        cid = lax.axis_index("c")
        sid = lax.axis_index("s")
        wid = cid * NS + sid

        pltpu.sync_copy(packed_h.at[wid], packed)
        pltpu.sync_copy(val_h.at[wid], vals)

        @pl.when(sid < WB_TILES)
        def _():
            pltpu.sync_copy(zero_h, acc.at[pl.ds(sid * WB_ROWS, WB_ROWS)])

        plsc.subcore_barrier()

        def unpack(ci, colb, rowb):
            for g in range(CHUNK // 16):
                sl = pl.ds(g * 16, 16)
                p = packed[pl.ds(ci * CHUNK + g * 16, 16)]
                colb[sl] = lax.bitwise_and(p, 0xFFFF)
                rowb[sl] = lax.shift_right_logical(p, 16)

        H = CHUNK // 2

        def gather_start(buf, colb, sema, semb):
            pltpu.async_copy(emb_h.at[colb.at[pl.ds(0, H)]],
                             buf.at[pl.ds(0, H)], sema)
            pltpu.async_copy(emb_h.at[colb.at[pl.ds(H, H)]],
                             buf.at[pl.ds(H, H)], semb)

        def gather_wait(buf, colb, sema, semb):
            pltpu.make_async_copy(emb_h.at[colb.at[pl.ds(0, H)]],
                                  buf.at[pl.ds(0, H)], sema).wait()
            pltpu.make_async_copy(emb_h.at[colb.at[pl.ds(H, H)]],
                                  buf.at[pl.ds(H, H)], semb).wait()

        def scatter_start(buf, rowb, sem):
            pltpu.async_copy(buf, acc.at[rowb], sem, add=True)

        def scatter_wait(buf, rowb, sem):
            pltpu.make_async_copy(buf, acc.at[rowb], sem).wait()

        def scale(buf, ci):
            for g in range(CHUNK // 16):
                vv = vals[pl.ds(ci * CHUNK + g * 16, 16)]
                for t in range(16):
                    v = vv[t]
                    e = g * 16 + t
                    for j in range(D // 16):
                        sl = pl.ds(j * 16, 16)
                        buf[e, sl] = buf[e, sl] * v

        unpack(0, colb0, rowb0)
        gather_start(buf0, colb0, gs0a, gs0b)
        unpack(1, colb1, rowb1)
        gather_start(buf1, colb1, gs1a, gs1b)

        def pair_body(i, carry):
            c0 = 2 * i
            c1 = 2 * i + 1
            gather_wait(buf0, colb0, gs0a, gs0b)
            scale(buf0, c0)
            scatter_start(buf0, rowb0, ss0)

            gather_wait(buf1, colb1, gs1a, gs1b)
            scale(buf1, c1)
            scatter_start(buf1, rowb1, ss1)

            scatter_wait(buf0, rowb0, ss0)
            unpack(c0 + 2, colb0, rowb0)
            gather_start(buf0, colb0, gs0a, gs0b)

            @pl.when(i < N_PAIRS - 1)
            def _():
                scatter_wait(buf1, rowb1, ss1)
                unpack(c1 + 2, colb1, rowb1)
                gather_start(buf1, colb1, gs1a, gs1b)

            return carry

        lax.fori_loop(0, N_PAIRS, pair_body, 0)

        last = N_CHUNKS - 1
        gather_wait(buf0, colb0, gs0a, gs0b)
        scale(buf0, last)
        scatter_start(buf0, rowb0, ss0)
        scatter_wait(buf1, rowb1, ss1)
        scatter_wait(buf0, rowb0, ss0)

        plsc.subcore_barrier()

        @pl.when(sid < WB_TILES)
        def _():
            sl = pl.ds(sid * WB_ROWS, WB_ROWS)
            pltpu.sync_copy(acc.at[sl], out_h.at[cid, sl])

    return k(packed3, val3, embeds, zeros_blk)


def _tc_add(partials):
    def body(p_ref, o_ref):
        o_ref[...] = p_ref[0] + p_ref[1]

    return pl.pallas_call(
        body,
        out_shape=jax.ShapeDtypeStruct((N, D), jnp.float32),
        grid=(10,),
        in_specs=[pl.BlockSpec((NC, N // 10, D), lambda i: (0, i, 0))],
        out_specs=pl.BlockSpec((N // 10, D), lambda i: (i, 0)),
    )(partials)


def kernel(edge_index, edge_values, embeds):
    row = edge_index[0].astype(jnp.int32)
    col = edge_index[1].astype(jnp.int32)
    packed3 = ((row << 16) | col).reshape(NW, E_PER_W)
    val3 = edge_values.reshape(NW, E_PER_W)
    zeros_blk = jnp.zeros((WB_ROWS, D), jnp.float32)
    partials = _sc_spmm(packed3, val3, embeds, zeros_blk)
    return _tc_add(partials)
